# baseline (speedup 1.0000x reference)
import jax
import jax.numpy as jnp
from jax.experimental import pallas as pl
from jax.experimental.pallas import tpu as pltpu

_VOCAB_TILE = 10240


def _body(ids_ref, emb_hbm, w_ref, out_ref, x_ref, xb_ref, sem):
    n_tok = x_ref.shape[0]
    seq = out_ref.shape[0]
    batch = out_ref.shape[2]

    @pl.when(pl.program_id(0) == 0)
    def _gather():
        def issue(i, c):
            r = (i % seq) * batch + i // seq
            pltpu.make_async_copy(
                emb_hbm.at[pl.ds(ids_ref[i], 1), :],
                x_ref.at[pl.ds(r, 1), :],
                sem,
            ).start()
            return c

        jax.lax.fori_loop(0, n_tok, issue, 0)
        pltpu.make_async_copy(
            emb_hbm.at[pl.ds(0, n_tok), :], x_ref, sem
        ).wait()
        xb_ref[...] = x_ref[...].astype(jnp.bfloat16)

    w = w_ref[...].astype(jnp.bfloat16)
    for s in range(seq):
        xs = xb_ref[s * batch : (s + 1) * batch, :]
        out_ref[s, :, :] = jax.lax.dot_general(
            w,
            xs,
            dimension_numbers=(((1,), (1,)), ((), ())),
            preferred_element_type=jnp.float32,
        )


def kernel(input_ids, embed_weight, lm_head_weight):
    batch, seq = input_ids.shape
    n_tok = batch * seq
    vocab, hidden = embed_weight.shape
    ids = input_ids.reshape(n_tok).astype(jnp.int32)

    n_tiles = pl.cdiv(vocab, _VOCAB_TILE)
    logits_svb = pl.pallas_call(
        _body,
        grid_spec=pltpu.PrefetchScalarGridSpec(
            num_scalar_prefetch=1,
            grid=(n_tiles,),
            in_specs=[
                pl.BlockSpec(memory_space=pl.ANY),
                pl.BlockSpec((_VOCAB_TILE, hidden), lambda j, ids_ref: (j, 0)),
            ],
            out_specs=pl.BlockSpec(
                (seq, _VOCAB_TILE, batch), lambda j, ids_ref: (0, j, 0)
            ),
            scratch_shapes=[
                pltpu.VMEM((n_tok, hidden), jnp.float32),
                pltpu.VMEM((n_tok, hidden), jnp.bfloat16),
                pltpu.SemaphoreType.DMA,
            ],
        ),
        out_shape=jax.ShapeDtypeStruct((seq, vocab, batch), jnp.float32),
    )(ids, embed_weight, lm_head_weight)

    return jnp.transpose(logits_svb, (2, 0, 1))


# speedup vs baseline: 1.0570x; 1.0570x over previous
import jax
import jax.numpy as jnp
from jax.experimental import pallas as pl
from jax.experimental.pallas import tpu as pltpu

_VOCAB_TILE = 10240


def _body(ids_ref, emb_hbm, w_ref, out_ref, x_ref, xb_ref, sem):
    n_tok = x_ref.shape[0]
    seq = out_ref.shape[0]
    batch = out_ref.shape[2]

    @pl.when(pl.program_id(0) == 0)
    def _gather():
        def issue(i, c):
            pltpu.make_async_copy(
                emb_hbm.at[pl.ds(ids_ref[i], 1), :],
                x_ref.at[pl.ds(i, 1), :],
                sem,
            ).start()
            return c

        jax.lax.fori_loop(0, n_tok, issue, 0)
        pltpu.make_async_copy(
            emb_hbm.at[pl.ds(0, n_tok), :], x_ref, sem
        ).wait()
        xb_ref[...] = x_ref[...].astype(jnp.bfloat16)

    w = w_ref[...].astype(jnp.bfloat16)
    for s in range(seq):
        xs = xb_ref[s * batch : (s + 1) * batch, :]
        out_ref[s, :, :] = jax.lax.dot_general(
            w,
            xs,
            dimension_numbers=(((1,), (1,)), ((), ())),
            preferred_element_type=jnp.float32,
        )


def kernel(input_ids, embed_weight, lm_head_weight):
    batch, seq = input_ids.shape
    n_tok = batch * seq
    vocab, hidden = embed_weight.shape
    ids = input_ids.T.reshape(n_tok).astype(jnp.int32)

    n_tiles = pl.cdiv(vocab, _VOCAB_TILE)
    logits_svb = pl.pallas_call(
        _body,
        grid_spec=pltpu.PrefetchScalarGridSpec(
            num_scalar_prefetch=1,
            grid=(n_tiles,),
            in_specs=[
                pl.BlockSpec(memory_space=pl.ANY),
                pl.BlockSpec((_VOCAB_TILE, hidden), lambda j, ids_ref: (j, 0)),
            ],
            out_specs=pl.BlockSpec(
                (seq, _VOCAB_TILE, batch), lambda j, ids_ref: (0, j, 0)
            ),
            scratch_shapes=[
                pltpu.VMEM((n_tok, hidden), jnp.float32),
                pltpu.VMEM((n_tok, hidden), jnp.bfloat16),
                pltpu.SemaphoreType.DMA,
            ],
        ),
        out_shape=jax.ShapeDtypeStruct((seq, vocab, batch), jnp.float32),
    )(ids, embed_weight, lm_head_weight)

    return jnp.transpose(logits_svb, (2, 0, 1))
